# baseline (speedup 1.0000x reference)
import functools

import jax
import jax.numpy as jnp
from jax import lax
from jax.experimental import pallas as pl
from jax.experimental.pallas import tpu as pltpu
from jax.experimental.pallas import tpu_sc as plsc

N_NODES = 10000
N_EDGES = 320000
D_FEAT = 128
HIDDEN = 16
N_CLASS = 40
F2 = 48

NC = 2
NS = 16
NW = NC * NS
EW = N_EDGES // NW
K = 80
NCHUNK = EW // K
NP = 10240
RPT = NP // NS


def _sc_mesh():
    return plsc.VectorSubcoreMesh(
        core_axis_name="c", subcore_axis_name="s", num_cores=NC, num_subcores=NS
    )


def _zero_rows(ref, nrows, ncols):
    z16 = jnp.zeros((16,), jnp.float32)

    def body(i, c):
        for j in range(ncols // 16):
            ref[i, pl.ds(16 * j, 16)] = z16
        return c

    lax.fori_loop(0, nrows, body, 0)


def _make_deg():
    @functools.partial(
        pl.kernel,
        out_type=jax.ShapeDtypeStruct((NC, NP, 16), jnp.float32),
        mesh=_sc_mesh(),
        compiler_params=pltpu.CompilerParams(use_tc_tiling_on_sc=False),
        scratch_types=[
            pltpu.VMEM((NCHUNK, K), jnp.int32),
            pltpu.VMEM((K, 16), jnp.float32),
            pltpu.VMEM((RPT, 16), jnp.float32),
            pltpu.VMEM_SHARED((NP, 16), jnp.float32),
        ],
    )
    def deg_kernel(dst_hbm, out_hbm, dst_v, ones_v, zbuf, acc):
        cid = lax.axis_index("c")
        sid = lax.axis_index("s")
        wid = sid * NC + cid

        _zero_rows(zbuf, RPT, 16)
        one16 = jnp.ones((16,), jnp.float32)

        def fill_ones(i, c):
            ones_v[i, :] = one16
            return c

        lax.fori_loop(0, K, fill_ones, 0)
        pltpu.sync_copy(zbuf, acc.at[pl.ds(sid * RPT, RPT)])
        pltpu.sync_copy(dst_hbm.at[wid], dst_v)
        plsc.subcore_barrier()

        def chunk(i, c):
            pltpu.sync_copy(ones_v, acc.at[dst_v.at[i]], add=True)
            return c

        lax.fori_loop(0, NCHUNK, chunk, 0)
        plsc.subcore_barrier()
        pltpu.sync_copy(
            acc.at[pl.ds(sid * RPT, RPT)],
            out_hbm.at[cid, pl.ds(sid * RPT, RPT)],
        )

    return deg_kernel


def _make_agg(F):
    @functools.partial(
        pl.kernel,
        out_type=jax.ShapeDtypeStruct((NC, NP, F), jnp.float32),
        mesh=_sc_mesh(),
        compiler_params=pltpu.CompilerParams(use_tc_tiling_on_sc=False),
        scratch_types=[
            pltpu.VMEM((NCHUNK, K), jnp.int32),
            pltpu.VMEM((NCHUNK, K), jnp.int32),
            pltpu.VMEM((2, K, F), jnp.float32),
            pltpu.VMEM((RPT, F), jnp.float32),
            pltpu.VMEM_SHARED((NP, F), jnp.float32),
            pltpu.SemaphoreType.DMA((2,)),
            pltpu.SemaphoreType.DMA((2,)),
        ],
    )
    def agg_kernel(
        y_hbm, src_hbm, dst_hbm, out_hbm, src_v, dst_v, buf, zbuf, acc, gsem, ssem
    ):
        cid = lax.axis_index("c")
        sid = lax.axis_index("s")
        wid = sid * NC + cid

        _zero_rows(zbuf, RPT, F)
        pltpu.sync_copy(zbuf, acc.at[pl.ds(sid * RPT, RPT)])
        pltpu.sync_copy(src_hbm.at[wid], src_v)
        pltpu.sync_copy(dst_hbm.at[wid], dst_v)
        plsc.subcore_barrier()

        pltpu.async_copy(y_hbm.at[src_v.at[0]], buf.at[0], gsem.at[0])

        def chunk(i, c):
            p = lax.rem(i, 2)
            pn = lax.rem(i + 1, 2)
            pltpu.make_async_copy(y_hbm.at[src_v.at[i]], buf.at[p], gsem.at[p]).wait()

            @pl.when(i > 0)
            def _():
                pltpu.make_async_copy(
                    buf.at[pn], acc.at[dst_v.at[i - 1]], ssem.at[pn]
                ).wait()

            @pl.when(i < NCHUNK - 1)
            def _():
                pltpu.async_copy(y_hbm.at[src_v.at[i + 1]], buf.at[pn], gsem.at[pn])

            pltpu.async_copy(buf.at[p], acc.at[dst_v.at[i]], ssem.at[p], add=True)
            return c

        lax.fori_loop(0, NCHUNK, chunk, 0)
        pf = (NCHUNK - 1) % 2
        pltpu.make_async_copy(
            buf.at[pf], acc.at[dst_v.at[NCHUNK - 1]], ssem.at[pf]
        ).wait()
        plsc.subcore_barrier()
        pltpu.sync_copy(
            acc.at[pl.ds(sid * RPT, RPT)],
            out_hbm.at[cid, pl.ds(sid * RPT, RPT)],
        )

    return agg_kernel


_R = 1000


def _tc1(x, W1, degp):
    def body(x_ref, w_ref, d_ref, y_ref, dinv_ref):
        xw = jnp.dot(x_ref[...], w_ref[...], preferred_element_type=jnp.float32)
        cnt = d_ref[0, :, 0:1] + d_ref[1, :, 0:1] + 1.0
        dinv = lax.rsqrt(cnt)
        y_ref[...] = xw * dinv
        dinv_ref[...] = jnp.broadcast_to(dinv, (_R, HIDDEN))

    return pl.pallas_call(
        body,
        grid=(N_NODES // _R,),
        in_specs=[
            pl.BlockSpec((_R, D_FEAT), lambda i: (i, 0)),
            pl.BlockSpec((D_FEAT, HIDDEN), lambda i: (0, 0)),
            pl.BlockSpec((NC, _R, 16), lambda i: (0, i, 0)),
        ],
        out_specs=[
            pl.BlockSpec((_R, HIDDEN), lambda i: (i, 0)),
            pl.BlockSpec((_R, HIDDEN), lambda i: (i, 0)),
        ],
        out_shape=[
            jax.ShapeDtypeStruct((N_NODES, HIDDEN), jnp.float32),
            jax.ShapeDtypeStruct((N_NODES, HIDDEN), jnp.float32),
        ],
    )(x, W1, degp)


def _tc2(agg1, y1, dinv, b1r, W2p):
    def body(a_ref, y_ref, dinv_ref, b_ref, w_ref, y2_ref):
        s = (a_ref[0] + a_ref[1] + y_ref[...]) * dinv_ref[...] + b_ref[...]
        h = jnp.maximum(s, 0.0)
        y2 = jnp.dot(h, w_ref[...], preferred_element_type=jnp.float32)
        y2_ref[...] = y2 * dinv_ref[...][:, 0:1]

    return pl.pallas_call(
        body,
        grid=(N_NODES // _R,),
        in_specs=[
            pl.BlockSpec((NC, _R, HIDDEN), lambda i: (0, i, 0)),
            pl.BlockSpec((_R, HIDDEN), lambda i: (i, 0)),
            pl.BlockSpec((_R, HIDDEN), lambda i: (i, 0)),
            pl.BlockSpec((1, HIDDEN), lambda i: (0, 0)),
            pl.BlockSpec((HIDDEN, F2), lambda i: (0, 0)),
        ],
        out_specs=pl.BlockSpec((_R, F2), lambda i: (i, 0)),
        out_shape=jax.ShapeDtypeStruct((N_NODES, F2), jnp.float32),
    )(agg1, y1, dinv, b1r, W2p)


def _tc3(agg2, y2, dinv, b2p):
    def body(a_ref, y_ref, dinv_ref, b_ref, o_ref):
        z = (a_ref[0] + a_ref[1] + y_ref[...]) * dinv_ref[...][:, 0:1] + b_ref[...]
        m = jnp.max(z, axis=1, keepdims=True)
        e = jnp.exp(z - m)
        ssum = jnp.sum(e, axis=1, keepdims=True)
        o_ref[...] = (z - m - jnp.log(ssum))[:, :N_CLASS]

    return pl.pallas_call(
        body,
        grid=(N_NODES // _R,),
        in_specs=[
            pl.BlockSpec((NC, _R, F2), lambda i: (0, i, 0)),
            pl.BlockSpec((_R, F2), lambda i: (i, 0)),
            pl.BlockSpec((_R, HIDDEN), lambda i: (i, 0)),
            pl.BlockSpec((1, F2), lambda i: (0, 0)),
        ],
        out_specs=pl.BlockSpec((_R, N_CLASS), lambda i: (i, 0)),
        out_shape=jax.ShapeDtypeStruct((N_NODES, N_CLASS), jnp.float32),
    )(agg2, y2, dinv, b2p)


def kernel(x, edge_index, W1, b1, W2, b2):
    src = edge_index[0].astype(jnp.int32).reshape(NW, NCHUNK, K)
    dst = edge_index[1].astype(jnp.int32).reshape(NW, NCHUNK, K)

    degp = _make_deg()(dst)
    y1, dinv = _tc1(x, W1, degp)
    agg1 = _make_agg(HIDDEN)(y1, src, dst)

    W2p = jnp.pad(W2, ((0, 0), (0, F2 - N_CLASS)))
    b1r = b1.reshape(1, HIDDEN)
    y2 = _tc2(agg1, y1, dinv, b1r, W2p)
    agg2 = _make_agg(F2)(y2, src, dst)

    b2p = jnp.concatenate(
        [b2, jnp.full((F2 - N_CLASS,), -1e30, jnp.float32)]
    ).reshape(1, F2)
    return _tc3(agg2, y2, dinv, b2p)


# speedup vs baseline: 1.5153x; 1.5153x over previous
import functools

import jax
import jax.numpy as jnp
from jax import lax
from jax.experimental import pallas as pl
from jax.experimental.pallas import tpu as pltpu
from jax.experimental.pallas import tpu_sc as plsc

N_NODES = 10000
N_EDGES = 320000
D_FEAT = 128
HIDDEN = 16
N_CLASS = 40
F2 = 48

NC = 2
NS = 16
NW = NC * NS
EW = N_EDGES // NW
K = 80
NCHUNK = EW // K
NP = 10240
RPT = NP // NS


def _sc_mesh():
    return plsc.VectorSubcoreMesh(
        core_axis_name="c", subcore_axis_name="s", num_cores=NC, num_subcores=NS
    )


def _zero_rows(ref, nrows, ncols):
    z16 = jnp.zeros((16,), jnp.float32)

    def body(i, c):
        for j in range(ncols // 16):
            ref[i, pl.ds(16 * j, 16)] = z16
        return c

    lax.fori_loop(0, nrows, body, 0)


def _make_deg():
    @functools.partial(
        pl.kernel,
        out_type=jax.ShapeDtypeStruct((NC, NP, 16), jnp.float32),
        mesh=_sc_mesh(),
        compiler_params=pltpu.CompilerParams(use_tc_tiling_on_sc=False),
        scratch_types=[
            pltpu.VMEM((NCHUNK, K), jnp.int32),
            pltpu.VMEM((K, 16), jnp.float32),
            pltpu.VMEM((RPT, 16), jnp.float32),
            pltpu.VMEM_SHARED((NP, 16), jnp.float32),
        ],
    )
    def deg_kernel(dst_hbm, out_hbm, dst_v, ones_v, zbuf, acc):
        cid = lax.axis_index("c")
        sid = lax.axis_index("s")
        wid = sid * NC + cid

        _zero_rows(zbuf, RPT, 16)
        one16 = jnp.ones((16,), jnp.float32)

        def fill_ones(i, c):
            ones_v[i, :] = one16
            return c

        lax.fori_loop(0, K, fill_ones, 0)
        pltpu.sync_copy(zbuf, acc.at[pl.ds(sid * RPT, RPT)])
        pltpu.sync_copy(dst_hbm.at[wid], dst_v)
        plsc.subcore_barrier()

        def chunk(i, c):
            pltpu.sync_copy(ones_v, acc.at[dst_v.at[i]], add=True)
            return c

        lax.fori_loop(0, NCHUNK, chunk, 0)
        plsc.subcore_barrier()
        pltpu.sync_copy(
            acc.at[pl.ds(sid * RPT, RPT)],
            out_hbm.at[cid, pl.ds(sid * RPT, RPT)],
        )

    return deg_kernel


def _make_agg(F):
    @functools.partial(
        pl.kernel,
        out_type=jax.ShapeDtypeStruct((NC, NP, F), jnp.float32),
        mesh=_sc_mesh(),
        compiler_params=pltpu.CompilerParams(use_tc_tiling_on_sc=False),
        scratch_types=[
            pltpu.VMEM((NCHUNK, K), jnp.int32),
            pltpu.VMEM((NCHUNK, K), jnp.int32),
            pltpu.VMEM((2, K, F), jnp.float32),
            pltpu.VMEM((RPT, F), jnp.float32),
            pltpu.VMEM_SHARED((NP, F), jnp.float32),
            pltpu.VMEM_SHARED((NP, F), jnp.float32),
            pltpu.SemaphoreType.DMA((2,)),
            pltpu.SemaphoreType.DMA((2,)),
        ],
    )
    def agg_kernel(
        y_hbm, src_hbm, dst_hbm, out_hbm, src_v, dst_v, buf, zbuf, acc, y_sp, gsem, ssem
    ):
        cid = lax.axis_index("c")
        sid = lax.axis_index("s")
        wid = sid * NC + cid

        yoff = jnp.minimum(sid * RPT, N_NODES - RPT)
        pltpu.sync_copy(
            y_hbm.at[pl.ds(yoff, RPT)], y_sp.at[pl.ds(yoff, RPT)]
        )
        _zero_rows(zbuf, RPT, F)
        pltpu.sync_copy(zbuf, acc.at[pl.ds(sid * RPT, RPT)])
        pltpu.sync_copy(src_hbm.at[wid], src_v)
        pltpu.sync_copy(dst_hbm.at[wid], dst_v)
        plsc.subcore_barrier()

        pltpu.async_copy(y_sp.at[src_v.at[0]], buf.at[0], gsem.at[0])

        def chunk(i, c):
            p = lax.rem(i, 2)
            pn = lax.rem(i + 1, 2)
            pltpu.make_async_copy(y_sp.at[src_v.at[i]], buf.at[p], gsem.at[p]).wait()

            @pl.when(i > 0)
            def _():
                pltpu.make_async_copy(
                    buf.at[pn], acc.at[dst_v.at[i - 1]], ssem.at[pn]
                ).wait()

            @pl.when(i < NCHUNK - 1)
            def _():
                pltpu.async_copy(y_sp.at[src_v.at[i + 1]], buf.at[pn], gsem.at[pn])

            pltpu.async_copy(buf.at[p], acc.at[dst_v.at[i]], ssem.at[p], add=True)
            return c

        lax.fori_loop(0, NCHUNK, chunk, 0)
        pf = (NCHUNK - 1) % 2
        pltpu.make_async_copy(
            buf.at[pf], acc.at[dst_v.at[NCHUNK - 1]], ssem.at[pf]
        ).wait()
        plsc.subcore_barrier()
        pltpu.sync_copy(
            acc.at[pl.ds(sid * RPT, RPT)],
            out_hbm.at[cid, pl.ds(sid * RPT, RPT)],
        )

    return agg_kernel


_R = 1000


def _tc1(x, W1, degp):
    def body(x_ref, w_ref, d_ref, y_ref, dinv_ref):
        xw = jnp.dot(x_ref[...], w_ref[...], preferred_element_type=jnp.float32)
        cnt = d_ref[0, :, 0:1] + d_ref[1, :, 0:1] + 1.0
        dinv = lax.rsqrt(cnt)
        y_ref[...] = xw * dinv
        dinv_ref[...] = jnp.broadcast_to(dinv, (_R, HIDDEN))

    return pl.pallas_call(
        body,
        grid=(N_NODES // _R,),
        in_specs=[
            pl.BlockSpec((_R, D_FEAT), lambda i: (i, 0)),
            pl.BlockSpec((D_FEAT, HIDDEN), lambda i: (0, 0)),
            pl.BlockSpec((NC, _R, 16), lambda i: (0, i, 0)),
        ],
        out_specs=[
            pl.BlockSpec((_R, HIDDEN), lambda i: (i, 0)),
            pl.BlockSpec((_R, HIDDEN), lambda i: (i, 0)),
        ],
        out_shape=[
            jax.ShapeDtypeStruct((N_NODES, HIDDEN), jnp.float32),
            jax.ShapeDtypeStruct((N_NODES, HIDDEN), jnp.float32),
        ],
    )(x, W1, degp)


def _tc2(agg1, y1, dinv, b1r, W2p):
    def body(a_ref, y_ref, dinv_ref, b_ref, w_ref, y2_ref):
        s = (a_ref[0] + a_ref[1] + y_ref[...]) * dinv_ref[...] + b_ref[...]
        h = jnp.maximum(s, 0.0)
        y2 = jnp.dot(h, w_ref[...], preferred_element_type=jnp.float32)
        y2_ref[...] = y2 * dinv_ref[...][:, 0:1]

    return pl.pallas_call(
        body,
        grid=(N_NODES // _R,),
        in_specs=[
            pl.BlockSpec((NC, _R, HIDDEN), lambda i: (0, i, 0)),
            pl.BlockSpec((_R, HIDDEN), lambda i: (i, 0)),
            pl.BlockSpec((_R, HIDDEN), lambda i: (i, 0)),
            pl.BlockSpec((1, HIDDEN), lambda i: (0, 0)),
            pl.BlockSpec((HIDDEN, F2), lambda i: (0, 0)),
        ],
        out_specs=pl.BlockSpec((_R, F2), lambda i: (i, 0)),
        out_shape=jax.ShapeDtypeStruct((N_NODES, F2), jnp.float32),
    )(agg1, y1, dinv, b1r, W2p)


def _tc3(agg2, y2, dinv, b2p):
    def body(a_ref, y_ref, dinv_ref, b_ref, o_ref):
        z = (a_ref[0] + a_ref[1] + y_ref[...]) * dinv_ref[...][:, 0:1] + b_ref[...]
        m = jnp.max(z, axis=1, keepdims=True)
        e = jnp.exp(z - m)
        ssum = jnp.sum(e, axis=1, keepdims=True)
        o_ref[...] = (z - m - jnp.log(ssum))[:, :N_CLASS]

    return pl.pallas_call(
        body,
        grid=(N_NODES // _R,),
        in_specs=[
            pl.BlockSpec((NC, _R, F2), lambda i: (0, i, 0)),
            pl.BlockSpec((_R, F2), lambda i: (i, 0)),
            pl.BlockSpec((_R, HIDDEN), lambda i: (i, 0)),
            pl.BlockSpec((1, F2), lambda i: (0, 0)),
        ],
        out_specs=pl.BlockSpec((_R, N_CLASS), lambda i: (i, 0)),
        out_shape=jax.ShapeDtypeStruct((N_NODES, N_CLASS), jnp.float32),
    )(agg2, y2, dinv, b2p)


def kernel(x, edge_index, W1, b1, W2, b2):
    src = edge_index[0].astype(jnp.int32).reshape(NW, NCHUNK, K)
    dst = edge_index[1].astype(jnp.int32).reshape(NW, NCHUNK, K)

    degp = _make_deg()(dst)
    y1, dinv = _tc1(x, W1, degp)
    agg1 = _make_agg(HIDDEN)(y1, src, dst)

    W2p = jnp.pad(W2, ((0, 0), (0, F2 - N_CLASS)))
    b1r = b1.reshape(1, HIDDEN)
    y2 = _tc2(agg1, y1, dinv, b1r, W2p)
    agg2 = _make_agg(F2)(y2, src, dst)

    b2p = jnp.concatenate(
        [b2, jnp.full((F2 - N_CLASS,), -1e30, jnp.float32)]
    ).reshape(1, F2)
    return _tc3(agg2, y2, dinv, b2p)
